# baseline (speedup 1.0000x reference)
import numpy as np
import jax
import jax.numpy as jnp
from jax.experimental import pallas as pl
from jax.experimental.pallas import tpu as pltpu

_B, _NV, _T = 8, 256, 2048
_VID_IN, _AUD_IN = 3 * 5 * 16 * 16, 2 * 16
_H = 256
_D = 256
_MASK_RATIO = 0.2
_NS = 4
_RS = _NV // _NS


def _static_mask():
    rng = np.random.RandomState(0)
    mask = np.zeros((_B, _T), dtype=bool)
    is_full = rng.rand(_B) < _MASK_RATIO
    for i in range(_B):
        if is_full[i]:
            if rng.randint(0, 2) == 1:
                mask[i, :_T // 2] = True
            else:
                mask[i, _T // 2:] = True
        else:
            S = int(_T * 0.2)
            pos = rng.permutation(_T)[:S]
            mask[i, pos] = True
    return mask


_MASK_NP = _static_mask()
_COUNTS = _MASK_NP.sum(axis=1)
_S_TOTAL = int(_COUNTS.sum())
_SEG = np.repeat(np.arange(_B), _COUNTS)
_EXPAND_NP = (np.arange(_B)[None, :] == _SEG[:, None]).astype(np.float32)

_SQ = 8 * _H
_BIAS0 = _SQ
_PACK_ROWS = _SQ + 10


def _body(*v_and_rest):
    v_refs = v_and_rest[:_NS]
    Wv_ref, pk_ref, ex_ref, out_ref, acc_ref, hv_ref = v_and_rest[_NS:]
    i = pl.program_id(0)

    sq = lambda k: pk_ref[pl.ds(k * _H, _H), :]
    brow = lambda j: pk_ref[pl.ds(_BIAS0 + j, 1), :]

    for j, vr in enumerate(v_refs):
        hv_ref[pl.ds(j * _RS, _RS), :] = jnp.dot(
            vr[0], Wv_ref[...], preferred_element_type=jnp.float32)
    hv = hv_ref[...] + brow(0)
    hv = jax.nn.relu(jnp.dot(hv, sq(0), preferred_element_type=jnp.float32)
                     + brow(1)) + hv
    hv = jax.nn.relu(jnp.dot(hv, sq(1), preferred_element_type=jnp.float32)
                     + brow(2)) + hv
    vis = jax.nn.relu(jnp.dot(hv, sq(2), preferred_element_type=jnp.float32)
                      + brow(3))
    acc_ref[pl.ds(i, 1), :] = (jnp.mean(vis, axis=0, keepdims=True)
                               + brow(4))

    @pl.when(i == _B - 1)
    def _tail():
        row = acc_ref[...]
        row = jax.nn.relu(jnp.dot(row, sq(3), preferred_element_type=jnp.float32)
                          + brow(5)) + row
        row = jax.nn.relu(jnp.dot(row, sq(4), preferred_element_type=jnp.float32)
                          + brow(6)) + row
        row = jax.nn.relu(jnp.dot(row, sq(5), preferred_element_type=jnp.float32)
                          + brow(7)) + row
        row = jax.nn.relu(jnp.dot(row, sq(6), preferred_element_type=jnp.float32)
                          + brow(8)) + row
        pred = (jnp.dot(row, sq(7), preferred_element_type=jnp.float32)
                + brow(9))[:, :_AUD_IN]
        out_ref[...] = jnp.dot(ex_ref[...], pred, preferred_element_type=jnp.float32)


def kernel(video, audio, params):
    del audio
    p = params
    row2 = lambda x: x.reshape(1, -1)
    padw = lambda x: jnp.pad(x, ((0, 0), (0, _H - x.shape[1])))

    packed = jnp.concatenate([
        p['vis'][0][0], p['vis'][1][0], p['W_sd_in'],
        p['sd'][0][0], p['sd'][1][0], p['ad'][0][0], p['ad'][1][0],
        padw(p['W_pred']),
        row2(p['b_v_in']), row2(p['vis'][0][1]), row2(p['vis'][1][1]),
        row2(p['b_sd_in']), row2(p['mask_embedding']),
        row2(p['sd'][0][1]), row2(p['sd'][1][1]),
        row2(p['ad'][0][1]), row2(p['ad'][1][1]),
        padw(row2(p['b_pred'])),
    ], axis=0)

    vspec = lambda j: pl.BlockSpec((1, _RS, _VID_IN), lambda i, j=j: (i, j, 0))
    full = lambda a: pl.BlockSpec(a.shape, lambda i: (0,) * a.ndim)
    expand = jnp.asarray(_EXPAND_NP)

    pred_audio = pl.pallas_call(
        _body,
        grid=(_B,),
        in_specs=([vspec(j) for j in range(_NS)]
                  + [full(p['W_v_in']), full(packed), full(expand)]),
        out_specs=pl.BlockSpec((_S_TOTAL, _AUD_IN), lambda i: (0, 0)),
        out_shape=jax.ShapeDtypeStruct((_S_TOTAL, _AUD_IN), jnp.float32),
        scratch_shapes=[pltpu.VMEM((_B, _D), jnp.float32),
                        pltpu.VMEM((_NV, _H), jnp.float32)],
    )(*([video] * _NS), p['W_v_in'], packed, expand)
    return (pred_audio, jnp.asarray(_MASK_NP))


# speedup vs baseline: 1.7708x; 1.7708x over previous
import numpy as np
import jax
import jax.numpy as jnp
from jax.experimental import pallas as pl
from jax.experimental.pallas import tpu as pltpu

_B, _NV, _T = 8, 256, 2048
_VID_IN, _AUD_IN = 3 * 5 * 16 * 16, 2 * 16
_H = 256
_D = 256
_MASK_RATIO = 0.2
_NS = 2
_RS = _NV // _NS


def _static_mask():
    rng = np.random.RandomState(0)
    mask = np.zeros((_B, _T), dtype=bool)
    is_full = rng.rand(_B) < _MASK_RATIO
    for i in range(_B):
        if is_full[i]:
            if rng.randint(0, 2) == 1:
                mask[i, :_T // 2] = True
            else:
                mask[i, _T // 2:] = True
        else:
            S = int(_T * 0.2)
            pos = rng.permutation(_T)[:S]
            mask[i, pos] = True
    return mask


_MASK_NP = _static_mask()
_COUNTS = _MASK_NP.sum(axis=1)
_S_TOTAL = int(_COUNTS.sum())
_SEG = np.repeat(np.arange(_B), _COUNTS)
_EXPAND_NP = (np.arange(_B)[None, :] == _SEG[:, None]).astype(np.float32)


def _body(*refs):
    v_refs = refs[:_NS]
    (Wv_ref, bv_ref, Wv1_ref, bv1_ref, Wv2_ref, bv2_ref,
     Wsd_ref, bsd_ref, me_ref,
     Ws1_ref, bs1_ref, Ws2_ref, bs2_ref,
     Wa1_ref, ba1_ref, Wa2_ref, ba2_ref,
     Wp_ref, bp_ref, ex_ref) = refs[_NS:_NS + 20]
    out_ref, acc_ref, hv_ref = refs[-3], refs[-2], refs[-1]
    i = pl.program_id(0)

    for j, vr in enumerate(v_refs):
        hv_ref[pl.ds(j * _RS, _RS), :] = jnp.dot(
            vr[0], Wv_ref[...], preferred_element_type=jnp.float32)
    hv = hv_ref[...] + bv_ref[...]
    hv = jax.nn.relu(jnp.dot(hv, Wv1_ref[...], preferred_element_type=jnp.float32)
                     + bv1_ref[...]) + hv
    hv = jax.nn.relu(jnp.dot(hv, Wv2_ref[...], preferred_element_type=jnp.float32)
                     + bv2_ref[...]) + hv
    vis = jax.nn.relu(jnp.dot(hv, Wsd_ref[...], preferred_element_type=jnp.float32)
                      + bsd_ref[...])
    acc_ref[pl.ds(i, 1), :] = (jnp.mean(vis, axis=0, keepdims=True)
                               + me_ref[...])

    @pl.when(i == _B - 1)
    def _tail():
        row = acc_ref[...]
        row = jax.nn.relu(jnp.dot(row, Ws1_ref[...], preferred_element_type=jnp.float32)
                          + bs1_ref[...]) + row
        row = jax.nn.relu(jnp.dot(row, Ws2_ref[...], preferred_element_type=jnp.float32)
                          + bs2_ref[...]) + row
        row = jax.nn.relu(jnp.dot(row, Wa1_ref[...], preferred_element_type=jnp.float32)
                          + ba1_ref[...]) + row
        row = jax.nn.relu(jnp.dot(row, Wa2_ref[...], preferred_element_type=jnp.float32)
                          + ba2_ref[...]) + row
        pred = jnp.dot(row, Wp_ref[...], preferred_element_type=jnp.float32) + bp_ref[...]
        out_ref[...] = jnp.dot(ex_ref[...], pred, preferred_element_type=jnp.float32)


def kernel(video, audio, params):
    del audio
    p = params
    row2 = lambda x: x.reshape(1, -1)
    full = lambda a: pl.BlockSpec(a.shape, lambda i: (0,) * a.ndim)

    weights = (
        p['W_v_in'], row2(p['b_v_in']),
        p['vis'][0][0], row2(p['vis'][0][1]),
        p['vis'][1][0], row2(p['vis'][1][1]),
        p['W_sd_in'], row2(p['b_sd_in']),
        row2(p['mask_embedding']),
        p['sd'][0][0], row2(p['sd'][0][1]),
        p['sd'][1][0], row2(p['sd'][1][1]),
        p['ad'][0][0], row2(p['ad'][0][1]),
        p['ad'][1][0], row2(p['ad'][1][1]),
        p['W_pred'], row2(p['b_pred']),
        jnp.asarray(_EXPAND_NP),
    )
    vspec = lambda j: pl.BlockSpec((1, _RS, _VID_IN), lambda i, j=j: (i, j, 0))
    in_specs = [vspec(j) for j in range(_NS)] + [full(a) for a in weights]

    pred_audio = pl.pallas_call(
        _body,
        grid=(_B,),
        in_specs=in_specs,
        out_specs=pl.BlockSpec((_S_TOTAL, _AUD_IN), lambda i: (0, 0)),
        out_shape=jax.ShapeDtypeStruct((_S_TOTAL, _AUD_IN), jnp.float32),
        scratch_shapes=[pltpu.VMEM((_B, _D), jnp.float32),
                        pltpu.VMEM((_NV, _H), jnp.float32)],
    )(*([video] * _NS), *weights)
    return (pred_audio, jnp.asarray(_MASK_NP))
